# baseline (speedup 1.0000x reference)
import functools

import jax
import jax.numpy as jnp
from jax import lax
from jax.experimental import pallas as pl
from jax.experimental.pallas import tpu as pltpu
from jax.experimental.pallas import tpu_sc as plsc

_NC = 2
_NS = 16
_NW = _NC * _NS
_LB = 32

_CH = 4
_NBUF = 4

_TW = 160


def _sc_gather_sums(table, idx_flat, n_pad, k, c):
    npw = n_pad // _NS
    nchunk = npw // _CH
    g = _CH * k
    nv = _TW // _LB
    nvf = c // _LB

    mesh = plsc.VectorSubcoreMesh(
        core_axis_name="c", subcore_axis_name="s",
        num_cores=1, num_subcores=_NS)

    @functools.partial(
        pl.kernel,
        out_type=(
            jax.ShapeDtypeStruct((n_pad, c), jnp.bfloat16),
            jax.ShapeDtypeStruct((n_pad, _LB), jnp.bfloat16),
        ),
        mesh=mesh,
        scratch_types=(
            [pltpu.VMEM((npw * k,), jnp.int32)]
            + [pltpu.VMEM((g, _TW), jnp.bfloat16) for _ in range(_NBUF)]
            + [pltpu.VMEM((npw, c), jnp.bfloat16),
               pltpu.VMEM((npw, _LB), jnp.bfloat16)]
            + [pltpu.SemaphoreType.DMA for _ in range(_NBUF)]
        ),
        compiler_params=pltpu.CompilerParams(use_tc_tiling_on_sc=False),
    )
    def sc_kernel(tbl_h, idx_h, outf_h, outc_h, idx_v, *scratch):
        rbs = list(scratch[:_NBUF])
        sumf_v, sumc_v = scratch[_NBUF], scratch[_NBUF + 1]
        sems = list(scratch[_NBUF + 2:2 * _NBUF + 2])
        sid = lax.axis_index("s")
        nbase = sid * npw

        pltpu.sync_copy(idx_h.at[pl.ds(nbase * k, npw * k)], idx_v)

        def start(chunk, b):
            pltpu.async_copy(tbl_h.at[idx_v.at[pl.ds(chunk * g, g)]],
                             rbs[b], sems[b])

        def wait(chunk, b):
            pltpu.make_async_copy(tbl_h.at[idx_v.at[pl.ds(chunk * g, g)]],
                                  rbs[b], sems[b]).wait()

        for b in range(_NBUF):
            start(b, b)

        @pl.loop(0, nchunk, step=_NBUF)
        def _chunks(c0):
            for b in range(_NBUF):
                ci = c0 + b
                wait(ci, b)
                for j in range(_CH):
                    r0 = j * k
                    acc0 = tuple(rbs[b][r0, pl.ds(v * _LB, _LB)]
                                 for v in range(nv))

                    def acc_step(kk, acc, b=b, r0=r0):
                        return tuple(
                            acc[v] + rbs[b][r0 + kk, pl.ds(v * _LB, _LB)]
                            for v in range(nv))

                    acc = pl.loop(1, k, init_carry=acc0, unroll=8)(acc_step)
                    node = ci * _CH + j
                    for v in range(nvf):
                        sumf_v[node, pl.ds(v * _LB, _LB)] = acc[v]
                    sumc_v[node, :] = acc[nvf]
                nxt = ci + _NBUF

                @pl.when(nxt < nchunk)
                def _():
                    start(nxt, b)

        pltpu.sync_copy(sumf_v, outf_h.at[pl.ds(nbase, npw)])
        pltpu.sync_copy(sumc_v, outc_h.at[pl.ds(nbase, npw)])

    return sc_kernel(table, idx_flat)


def _tc_dense(feat, sums_f, sums_c, c8, w1, w2k, w3m, w3s, b2, inv_k, br):
    n, c = feat.shape

    def body(f_ref, sf_ref, sc_ref, c8_ref, w1_ref, w2_ref, w3m_ref,
             w3s_ref, b_ref, o_ref):
        f = f_ref[...]
        sf = sf_ref[...].astype(jnp.float32)
        q = sc_ref[...].astype(jnp.float32) * inv_k
        q1 = q[:, :8]
        q2 = q[:, 8:16]
        rel_m = q1 - c8_ref[...]
        var = jnp.maximum(q2 - q1 * q1, 0.0)
        rel_s = jnp.sqrt(var)
        acc = jnp.dot(f, w1_ref[...], preferred_element_type=jnp.float32)
        acc += jnp.dot(sf, w2_ref[...], preferred_element_type=jnp.float32)
        acc += jnp.dot(rel_m, w3m_ref[...], preferred_element_type=jnp.float32)
        acc += jnp.dot(rel_s, w3s_ref[...], preferred_element_type=jnp.float32)
        acc += b_ref[...]
        o_ref[...] = jnp.maximum(acc, 0.0)

    nb = n // br
    row = lambda i: (i, 0)
    fixed = lambda i: (0, 0)
    return pl.pallas_call(
        body,
        grid=(nb,),
        in_specs=[
            pl.BlockSpec((br, c), row),
            pl.BlockSpec((br, c), row),
            pl.BlockSpec((br, _LB), row),
            pl.BlockSpec((br, 8), row),
            pl.BlockSpec((c, c), fixed),
            pl.BlockSpec((c, c), fixed),
            pl.BlockSpec((8, c), fixed),
            pl.BlockSpec((8, c), fixed),
            pl.BlockSpec((1, c), fixed),
        ],
        out_specs=pl.BlockSpec((br, c), row),
        out_shape=jax.ShapeDtypeStruct((n, c), jnp.float32),
    )(feat, sums_f, sums_c, c8, w1, w2k, w3m, w3s, b2)


def kernel(feat, coords, knn_idx, W, b):
    n, c = feat.shape
    k = knn_idx.shape[1]
    inv_k = 1.0 / k

    chunk_nodes = _NS * _CH * _NBUF
    n_pad = ((n + chunk_nodes - 1) // chunk_nodes) * chunk_nodes

    c8 = jnp.pad(coords, ((0, 0), (0, 8 - coords.shape[1])))
    table = jnp.concatenate(
        [feat, c8, c8 * c8, jnp.zeros((n, 16), jnp.float32)],
        axis=1).astype(jnp.bfloat16)

    idx_flat = jnp.pad(knn_idx.reshape(-1), (0, (n_pad - n) * k))

    sums_f, sums_c = _sc_gather_sums(table, idx_flat, n_pad, k, c)

    w1 = W[:c]
    w2k = W[c:2 * c] * inv_k
    w3m = jnp.zeros((8, c), jnp.float32).at[:3].set(W[2 * c:2 * c + 3])
    w3s = jnp.zeros((8, c), jnp.float32).at[:3].set(W[2 * c + 3:2 * c + 6])
    b2 = b.reshape(1, c)

    br = 1000 if n % 1000 == 0 else 8
    return _tc_dense(feat, sums_f[:n], sums_c[:n], c8, w1, w2k, w3m, w3s,
                     b2, inv_k, br)


# speedup vs baseline: 1.0109x; 1.0109x over previous
import functools

import jax
import jax.numpy as jnp
from jax import lax
from jax.experimental import pallas as pl
from jax.experimental.pallas import tpu as pltpu
from jax.experimental.pallas import tpu_sc as plsc

_NC = 2
_NS = 16
_NW = _NC * _NS
_LB = 32

_CH = 4
_NBUF = 4
_NPASS = 2

_TW = 160


def _sc_gather_sums(table, idx_flat, n_pad, k, c):
    npw = n_pad // _NS
    npp = npw // _NPASS
    nchunk = npp // _CH
    g = _CH * k
    nv = _TW // _LB
    nvf = c // _LB

    mesh = plsc.VectorSubcoreMesh(
        core_axis_name="c", subcore_axis_name="s",
        num_cores=1, num_subcores=_NS)

    @functools.partial(
        pl.kernel,
        out_type=(
            jax.ShapeDtypeStruct((n_pad, c), jnp.bfloat16),
            jax.ShapeDtypeStruct((n_pad, _LB), jnp.bfloat16),
        ),
        mesh=mesh,
        scratch_types=(
            [pltpu.VMEM((npp * k,), jnp.int32)]
            + [pltpu.VMEM((g, _TW), jnp.bfloat16) for _ in range(_NBUF)]
            + [pltpu.VMEM((npp, c), jnp.bfloat16),
               pltpu.VMEM((npp, _LB), jnp.bfloat16)]
            + [pltpu.SemaphoreType.DMA for _ in range(_NBUF)]
        ),
        compiler_params=pltpu.CompilerParams(use_tc_tiling_on_sc=False),
    )
    def sc_kernel(tbl_h, idx_h, outf_h, outc_h, idx_v, *scratch):
        rbs = list(scratch[:_NBUF])
        sumf_v, sumc_v = scratch[_NBUF], scratch[_NBUF + 1]
        sems = list(scratch[_NBUF + 2:2 * _NBUF + 2])
        sid = lax.axis_index("s")

        @pl.loop(0, _NPASS)
        def _passes(p):
            nbase = sid * npw + p * npp

            pltpu.sync_copy(idx_h.at[pl.ds(nbase * k, npp * k)], idx_v)

            def start(chunk, b):
                pltpu.async_copy(tbl_h.at[idx_v.at[pl.ds(chunk * g, g)]],
                                 rbs[b], sems[b])

            def wait(chunk, b):
                pltpu.make_async_copy(tbl_h.at[idx_v.at[pl.ds(chunk * g, g)]],
                                      rbs[b], sems[b]).wait()

            for b in range(_NBUF):
                start(b, b)

            @pl.loop(0, nchunk, step=_NBUF)
            def _chunks(c0):
                for b in range(_NBUF):
                    ci = c0 + b
                    wait(ci, b)
                    for j in range(_CH):
                        r0 = j * k
                        acc0 = tuple(rbs[b][r0, pl.ds(v * _LB, _LB)]
                                     for v in range(nv))

                        def acc_step(kk, acc, b=b, r0=r0):
                            return tuple(
                                acc[v] + rbs[b][r0 + kk, pl.ds(v * _LB, _LB)]
                                for v in range(nv))

                        acc = pl.loop(1, k, init_carry=acc0,
                                      unroll=8)(acc_step)
                        node = ci * _CH + j
                        for v in range(nvf):
                            sumf_v[node, pl.ds(v * _LB, _LB)] = acc[v]
                        sumc_v[node, :] = acc[nvf]
                    nxt = ci + _NBUF

                    @pl.when(nxt < nchunk)
                    def _():
                        start(nxt, b)

            pltpu.sync_copy(sumf_v, outf_h.at[pl.ds(nbase, npp)])
            pltpu.sync_copy(sumc_v, outc_h.at[pl.ds(nbase, npp)])

    return sc_kernel(table, idx_flat)


def _tc_dense(feat, sums_f, sums_c, c8, w1, w2k, w3m, w3s, b2, inv_k, br):
    n, c = feat.shape

    def body(f_ref, sf_ref, sc_ref, c8_ref, w1_ref, w2_ref, w3m_ref,
             w3s_ref, b_ref, o_ref):
        f = f_ref[...]
        sf = sf_ref[...].astype(jnp.float32)
        q = sc_ref[...].astype(jnp.float32) * inv_k
        q1 = q[:, :8]
        q2 = q[:, 8:16]
        rel_m = q1 - c8_ref[...]
        var = jnp.maximum(q2 - q1 * q1, 0.0)
        rel_s = jnp.sqrt(var)
        acc = jnp.dot(f, w1_ref[...], preferred_element_type=jnp.float32)
        acc += jnp.dot(sf, w2_ref[...], preferred_element_type=jnp.float32)
        acc += jnp.dot(rel_m, w3m_ref[...], preferred_element_type=jnp.float32)
        acc += jnp.dot(rel_s, w3s_ref[...], preferred_element_type=jnp.float32)
        acc += b_ref[...]
        o_ref[...] = jnp.maximum(acc, 0.0)

    nb = n // br
    row = lambda i: (i, 0)
    fixed = lambda i: (0, 0)
    return pl.pallas_call(
        body,
        grid=(nb,),
        in_specs=[
            pl.BlockSpec((br, c), row),
            pl.BlockSpec((br, c), row),
            pl.BlockSpec((br, _LB), row),
            pl.BlockSpec((br, 8), row),
            pl.BlockSpec((c, c), fixed),
            pl.BlockSpec((c, c), fixed),
            pl.BlockSpec((8, c), fixed),
            pl.BlockSpec((8, c), fixed),
            pl.BlockSpec((1, c), fixed),
        ],
        out_specs=pl.BlockSpec((br, c), row),
        out_shape=jax.ShapeDtypeStruct((n, c), jnp.float32),
    )(feat, sums_f, sums_c, c8, w1, w2k, w3m, w3s, b2)


def kernel(feat, coords, knn_idx, W, b):
    n, c = feat.shape
    k = knn_idx.shape[1]
    inv_k = 1.0 / k

    chunk_nodes = _NS * _CH * _NBUF * _NPASS
    n_pad = ((n + chunk_nodes - 1) // chunk_nodes) * chunk_nodes

    c8 = jnp.pad(coords, ((0, 0), (0, 8 - coords.shape[1])))
    table = jnp.concatenate(
        [feat.astype(jnp.bfloat16), c8.astype(jnp.bfloat16),
         (c8 * c8).astype(jnp.bfloat16), jnp.zeros((n, 16), jnp.bfloat16)],
        axis=1)

    idx_flat = jnp.pad(knn_idx.reshape(-1), (0, (n_pad - n) * k))

    sums_f, sums_c = _sc_gather_sums(table, idx_flat, n_pad, k, c)

    w1 = W[:c]
    w2k = W[c:2 * c] * inv_k
    w3m = jnp.zeros((8, c), jnp.float32).at[:3].set(W[2 * c:2 * c + 3])
    w3s = jnp.zeros((8, c), jnp.float32).at[:3].set(W[2 * c + 3:2 * c + 6])
    b2 = b.reshape(1, c)

    br = 1000 if n % 1000 == 0 else 8
    return _tc_dense(feat, sums_f[:n], sums_c[:n], c8, w1, w2k, w3m, w3s,
                     b2, inv_k, br)


# speedup vs baseline: 1.1083x; 1.0963x over previous
import functools

import jax
import jax.numpy as jnp
from jax import lax
from jax.experimental import pallas as pl
from jax.experimental.pallas import tpu as pltpu
from jax.experimental.pallas import tpu_sc as plsc

_NC = 2
_NS = 16
_NW = _NC * _NS
_LB = 32

_CH = 4
_NBUF = 4

_TW = 160


def _sc_gather_sums(table, idx_flat, n_pad, k, c):
    npw0 = (n_pad * 3 // 4) // _NS
    npw1 = (n_pad // 4) // _NS
    g = _CH * k
    nv = _TW // _LB
    nvf = c // _LB

    mesh = plsc.VectorSubcoreMesh(
        core_axis_name="c", subcore_axis_name="s",
        num_cores=_NC, num_subcores=_NS)

    @functools.partial(
        pl.kernel,
        out_type=(
            jax.ShapeDtypeStruct((n_pad, c), jnp.bfloat16),
            jax.ShapeDtypeStruct((n_pad, _LB), jnp.bfloat16),
        ),
        mesh=mesh,
        scratch_types=(
            [pltpu.VMEM((npw0 * k,), jnp.int32)]
            + [pltpu.VMEM((g, _TW), jnp.bfloat16) for _ in range(_NBUF)]
            + [pltpu.VMEM((npw0, c), jnp.bfloat16),
               pltpu.VMEM((npw0, _LB), jnp.bfloat16)]
            + [pltpu.SemaphoreType.DMA for _ in range(_NBUF)]
        ),
        compiler_params=pltpu.CompilerParams(use_tc_tiling_on_sc=False),
    )
    def sc_kernel(tbl_h, idx_h, outf_h, outc_h, idx_v, *scratch):
        rbs = list(scratch[:_NBUF])
        sumf_v, sumc_v = scratch[_NBUF], scratch[_NBUF + 1]
        sems = list(scratch[_NBUF + 2:2 * _NBUF + 2])
        cid = lax.axis_index("c")
        sid = lax.axis_index("s")

        def run(npw, nbase):
            nchunk = npw // _CH

            pltpu.sync_copy(idx_h.at[pl.ds(nbase * k, npw * k)],
                            idx_v.at[pl.ds(0, npw * k)])

            def start(chunk, b):
                pltpu.async_copy(tbl_h.at[idx_v.at[pl.ds(chunk * g, g)]],
                                 rbs[b], sems[b])

            def wait(chunk, b):
                pltpu.make_async_copy(tbl_h.at[idx_v.at[pl.ds(chunk * g, g)]],
                                      rbs[b], sems[b]).wait()

            for b in range(_NBUF):
                start(b, b)

            @pl.loop(0, nchunk, step=_NBUF)
            def _chunks(c0):
                for b in range(_NBUF):
                    ci = c0 + b
                    wait(ci, b)
                    for j in range(_CH):
                        r0 = j * k
                        acc = [rbs[b][r0, pl.ds(v * _LB, _LB)]
                               for v in range(nv)]
                        for kk in range(1, k):
                            for v in range(nv):
                                acc[v] = acc[v] + rbs[b][r0 + kk,
                                                         pl.ds(v * _LB, _LB)]
                        node = ci * _CH + j
                        for v in range(nvf):
                            sumf_v[node, pl.ds(v * _LB, _LB)] = acc[v]
                        sumc_v[node, :] = acc[nvf]
                    nxt = ci + _NBUF

                    @pl.when(nxt < nchunk)
                    def _():
                        start(nxt, b)

            pltpu.sync_copy(sumf_v.at[pl.ds(0, npw)],
                            outf_h.at[pl.ds(nbase, npw)])
            pltpu.sync_copy(sumc_v.at[pl.ds(0, npw)],
                            outc_h.at[pl.ds(nbase, npw)])

        @pl.when(cid == 0)
        def _():
            run(npw0, sid * npw0)

        @pl.when(cid == 1)
        def _():
            run(npw1, _NS * npw0 + sid * npw1)

    return sc_kernel(table, idx_flat)


def _tc_dense(feat, sums_f, sums_c, c8, w1, w2k, w3m, w3s, b2, inv_k, br):
    n, c = feat.shape

    def body(f_ref, sf_ref, sc_ref, c8_ref, w1_ref, w2_ref, w3m_ref,
             w3s_ref, b_ref, o_ref):
        f = f_ref[...]
        sf = sf_ref[...].astype(jnp.float32)
        q = sc_ref[...].astype(jnp.float32) * inv_k
        q1 = q[:, :8]
        q2 = q[:, 8:16]
        rel_m = q1 - c8_ref[...]
        var = jnp.maximum(q2 - q1 * q1, 0.0)
        rel_s = jnp.sqrt(var)
        acc = jnp.dot(f, w1_ref[...], preferred_element_type=jnp.float32)
        acc += jnp.dot(sf, w2_ref[...], preferred_element_type=jnp.float32)
        acc += jnp.dot(rel_m, w3m_ref[...], preferred_element_type=jnp.float32)
        acc += jnp.dot(rel_s, w3s_ref[...], preferred_element_type=jnp.float32)
        acc += b_ref[...]
        o_ref[...] = jnp.maximum(acc, 0.0)

    nb = n // br
    row = lambda i: (i, 0)
    fixed = lambda i: (0, 0)
    return pl.pallas_call(
        body,
        grid=(nb,),
        in_specs=[
            pl.BlockSpec((br, c), row),
            pl.BlockSpec((br, c), row),
            pl.BlockSpec((br, _LB), row),
            pl.BlockSpec((br, 8), row),
            pl.BlockSpec((c, c), fixed),
            pl.BlockSpec((c, c), fixed),
            pl.BlockSpec((8, c), fixed),
            pl.BlockSpec((8, c), fixed),
            pl.BlockSpec((1, c), fixed),
        ],
        out_specs=pl.BlockSpec((br, c), row),
        out_shape=jax.ShapeDtypeStruct((n, c), jnp.float32),
    )(feat, sums_f, sums_c, c8, w1, w2k, w3m, w3s, b2)


def kernel(feat, coords, knn_idx, W, b):
    n, c = feat.shape
    k = knn_idx.shape[1]
    inv_k = 1.0 / k

    chunk_nodes = _NW * _CH * _NBUF * 2
    n_pad = ((n + chunk_nodes - 1) // chunk_nodes) * chunk_nodes

    c8 = jnp.pad(coords, ((0, 0), (0, 8 - coords.shape[1])))
    table = jnp.concatenate(
        [feat.astype(jnp.bfloat16), c8.astype(jnp.bfloat16),
         (c8 * c8).astype(jnp.bfloat16), jnp.zeros((n, 16), jnp.bfloat16)],
        axis=1)

    n_idx = n_pad + n_pad // 32
    idx_flat = jnp.pad(knn_idx.reshape(-1), (0, (n_idx - n) * k))

    sums_f, sums_c = _sc_gather_sums(table, idx_flat, n_pad, k, c)

    w1 = W[:c]
    w2k = W[c:2 * c] * inv_k
    w3m = jnp.zeros((8, c), jnp.float32).at[:3].set(W[2 * c:2 * c + 3])
    w3s = jnp.zeros((8, c), jnp.float32).at[:3].set(W[2 * c + 3:2 * c + 6])
    b2 = b.reshape(1, c)

    br = 1000 if n % 1000 == 0 else 8
    return _tc_dense(feat, sums_f[:n], sums_c[:n], c8, w1, w2k, w3m, w3s,
                     b2, inv_k, br)
